# baseline (speedup 1.0000x reference)
import functools

import jax
import jax.numpy as jnp
from jax.experimental import pallas as pl
from jax.experimental.pallas import tpu as pltpu

EMB = 256
CTX = 2048
K = 4
HIDDEN = 1024
NH = 4
GADD = 2
NADD = 2
RANK = 2
NUM_POINTS = K * (2 ** RANK + GADD + NADD)
SIGMA_BOOST = 2.0
EPS = 1e-7


def _softplus(v):
    return jnp.logaddexp(v, 0.0)


def _proj_body(x_ref, wqkv_ref, wp1_ref, bp1_ref, wp2_ref, bp2_ref,
               qkv_ref, params_ref):
    x = x_ref[...]
    qkv_ref[...] = jnp.dot(x, wqkv_ref[...], preferred_element_type=jnp.float32)
    h = jnp.maximum(
        jnp.dot(x, wp1_ref[...], preferred_element_type=jnp.float32)
        + bp1_ref[...], 0.0)
    params_ref[...] = (
        jnp.dot(h, wp2_ref[...], preferred_element_type=jnp.float32)
        + bp2_ref[...])


def _dense_projections(x2d, Wq, Wk, Wv, Wp1, bp1, Wp2, bp2):
    wqkv = jnp.concatenate([Wq, Wk, Wv], axis=1)
    bp2p = jnp.pad(bp2, (0, 128 - 2 * K))
    wp2p = jnp.pad(Wp2, ((0, 0), (0, 128 - 2 * K)))
    qkv, params = pl.pallas_call(
        _proj_body,
        out_shape=(
            jax.ShapeDtypeStruct((CTX, 3 * EMB * NH), jnp.float32),
            jax.ShapeDtypeStruct((CTX, 128), jnp.float32),
        ),
    )(x2d, wqkv, Wp1, bp1[None, :], wp2p, bp2p[None, :])
    return qkv, params[:, : 2 * K]


def _out_body(y_ref, wu_ref, bu_ref, o_ref):
    o_ref[...] = (
        jnp.dot(y_ref[...], wu_ref[...], preferred_element_type=jnp.float32)
        + bu_ref[...])


def _out_projection(y, Wu, bu):
    return pl.pallas_call(
        _out_body,
        out_shape=jax.ShapeDtypeStruct((CTX, EMB), jnp.float32),
    )(y, Wu, bu[None, :])


def _gen_indices(params):
    c = CTX
    diags = jnp.broadcast_to(
        jnp.arange(c, dtype=jnp.float32)[:, None, None], (c, K, RANK))
    means = params[:, :K].reshape(c, K, 1)
    sigmas = params[:, K:].reshape(c, K)
    means = diags - _softplus(means)
    means = jax.nn.sigmoid(means) * (c - 1)
    sigmas = (_softplus(sigmas + SIGMA_BOOST) + EPS)[..., None] * (c - 1)

    fm = jnp.floor(means)
    offs = jnp.array([[0.0, 0.0], [0.0, 1.0], [1.0, 0.0], [1.0, 1.0]],
                     dtype=jnp.float32)
    neigh = fm[..., None, :] + offs[None, None]
    gk, nk = jax.random.split(jax.random.key(42))
    glob = jax.random.randint(
        gk, (1, c, K, GADD, RANK), 0, c).astype(jnp.float32)[0]
    local = fm[..., None, :] - 1.0 + jax.random.randint(
        nk, (1, c, K, NADD, RANK), 0, 2).astype(jnp.float32)[0]
    pts = jnp.concatenate([neigh, glob, local], axis=2)
    pts = jnp.clip(pts, 0.0, float(c - 1))
    indices = pts.reshape(c, NUM_POINTS, RANK).astype(jnp.int32)

    ifl = indices.astype(jnp.float32)
    m = means[:, None, :, :]
    s = sigmas[:, None, :, :]
    inv = jnp.sqrt(1.0 / (EPS + s * s))
    diff = (ifl[:, :, None, :] - m) * inv
    dens = jnp.exp(-0.5 * jnp.sum(diff * diff, axis=-1))

    code = indices[..., 0] * c + indices[..., 1]
    eq = code[:, :, None] == code[:, None, :]
    lower = jnp.tril(jnp.ones((NUM_POINTS, NUM_POINTS), dtype=bool), -1)
    dup = jnp.any(eq & lower[None], axis=-1)
    dens = jnp.where(dup[..., None], 0.0, dens)
    dens = dens / jnp.sum(dens, axis=1, keepdims=True)
    weights = jnp.sum(dens, axis=2)
    return indices, weights


def _qk_body(q_ref, kt_ref, a_ref):
    a_ref[...] = jnp.dot(q_ref[0], kt_ref[0],
                         preferred_element_type=jnp.float32)[None]


def _qk_matmul(queries, keys):
    kt = keys.transpose(0, 2, 1)
    return pl.pallas_call(
        _qk_body,
        grid=(NH,),
        in_specs=[
            pl.BlockSpec((1, CTX, EMB), lambda h: (h, 0, 0)),
            pl.BlockSpec((1, EMB, CTX), lambda h: (h, 0, 0)),
        ],
        out_specs=pl.BlockSpec((1, CTX, CTX), lambda h: (h, 0, 0)),
        out_shape=jax.ShapeDtypeStruct((NH, CTX, CTX), jnp.float32),
    )(queries, kt)


def _pv_body(p_ref, v_ref, o_ref):
    p = p_ref[0]
    s = jnp.sum(p, axis=1, keepdims=True)
    p = p / (s + EPS)
    o_ref[...] = jnp.dot(p, v_ref[0], preferred_element_type=jnp.float32)[None]


def _pv_matmul(p, values):
    return pl.pallas_call(
        _pv_body,
        grid=(NH,),
        in_specs=[
            pl.BlockSpec((1, CTX, CTX), lambda h: (h, 0, 0)),
            pl.BlockSpec((1, CTX, EMB), lambda h: (h, 0, 0)),
        ],
        out_specs=pl.BlockSpec((1, CTX, EMB), lambda h: (h, 0, 0)),
        out_shape=jax.ShapeDtypeStruct((NH, CTX, EMB), jnp.float32),
    )(p, values)


def _sparse_attn(qkv, indices, weights):
    c = CTX
    scale = EMB ** 0.25
    q, k, v = jnp.split(qkv, 3, axis=1)

    def split_heads(t):
        return t.reshape(c, NH, EMB).transpose(1, 0, 2)

    queries = split_heads(q) / scale
    keys = split_heads(k) / scale
    values = split_heads(v)

    idx = indices.reshape(c * NUM_POINTS, RANK)
    w = weights.reshape(c * NUM_POINTS)
    rows = idx[:, 0]
    cols = idx[:, 1]
    code = rows * c + cols

    a_full = _qk_matmul(queries, keys)
    a_t = a_full.reshape(NH, c * c).T
    a = jnp.take(a_t, code, axis=0)
    val = w[:, None] * a

    mx = jax.ops.segment_max(val, rows, num_segments=c)
    mx = jnp.where(jnp.isfinite(mx), mx, 0.0)
    ex = jnp.exp(val - jnp.take(mx, rows, axis=0))

    p = jnp.zeros((c * c, NH), jnp.float32)
    p = p.at[code].add(ex)
    out = _pv_matmul(p.T.reshape(NH, c, c), values)
    return out.transpose(1, 0, 2).reshape(c, NH * EMB)


@jax.jit
def kernel(x, Wq, Wk, Wv, Wu, bu, Wp1, bp1, Wp2, bp2):
    x2d = x[0]
    qkv, params = _dense_projections(x2d, Wq, Wk, Wv, Wp1, bp1, Wp2, bp2)
    indices, weights = _gen_indices(params)
    y = _sparse_attn(qkv, indices, weights)
    out = _out_projection(y, Wu, bu)
    return out[None]


# speedup vs baseline: 2.3151x; 2.3151x over previous
import functools

import jax
import jax.numpy as jnp
from jax.experimental import pallas as pl
from jax.experimental.pallas import tpu as pltpu

EMB = 256
CTX = 2048
K = 4
HIDDEN = 1024
NH = 4
GADD = 2
NADD = 2
RANK = 2
NUM_POINTS = K * (2 ** RANK + GADD + NADD)
SIGMA_BOOST = 2.0
EPS = 1e-7


def _softplus(v):
    return jnp.logaddexp(v, 0.0)


def _proj_body(x_ref, wqkv_ref, wp1_ref, bp1_ref, wp2_ref, bp2_ref,
               qkv_ref, params_ref):
    x = x_ref[...]
    qkv_ref[...] = jnp.dot(x, wqkv_ref[...], preferred_element_type=jnp.float32)
    h = jnp.maximum(
        jnp.dot(x, wp1_ref[...], preferred_element_type=jnp.float32)
        + bp1_ref[...], 0.0)
    params_ref[...] = (
        jnp.dot(h, wp2_ref[...], preferred_element_type=jnp.float32)
        + bp2_ref[...])


def _dense_projections(x2d, Wq, Wk, Wv, Wp1, bp1, Wp2, bp2):
    wqkv = jnp.concatenate([Wq, Wk, Wv], axis=1)
    bp2p = jnp.pad(bp2, (0, 128 - 2 * K))
    wp2p = jnp.pad(Wp2, ((0, 0), (0, 128 - 2 * K)))
    qkv, params = pl.pallas_call(
        _proj_body,
        out_shape=(
            jax.ShapeDtypeStruct((CTX, 3 * EMB * NH), jnp.float32),
            jax.ShapeDtypeStruct((CTX, 128), jnp.float32),
        ),
    )(x2d, wqkv, Wp1, bp1[None, :], wp2p, bp2p[None, :])
    return qkv, params[:, : 2 * K]


def _out_body(y_ref, wu_ref, bu_ref, o_ref):
    o_ref[...] = (
        jnp.dot(y_ref[...], wu_ref[...], preferred_element_type=jnp.float32)
        + bu_ref[...])


def _out_projection(y, Wu, bu):
    return pl.pallas_call(
        _out_body,
        out_shape=jax.ShapeDtypeStruct((CTX, EMB), jnp.float32),
    )(y, Wu, bu[None, :])


def _gen_indices(params):
    c = CTX
    diags = jnp.broadcast_to(
        jnp.arange(c, dtype=jnp.float32)[:, None, None], (c, K, RANK))
    means = params[:, :K].reshape(c, K, 1)
    sigmas = params[:, K:].reshape(c, K)
    means = diags - _softplus(means)
    means = jax.nn.sigmoid(means) * (c - 1)
    sigmas = (_softplus(sigmas + SIGMA_BOOST) + EPS)[..., None] * (c - 1)

    fm = jnp.floor(means)
    offs = jnp.array([[0.0, 0.0], [0.0, 1.0], [1.0, 0.0], [1.0, 1.0]],
                     dtype=jnp.float32)
    neigh = fm[..., None, :] + offs[None, None]
    gk, nk = jax.random.split(jax.random.key(42))
    glob = jax.random.randint(
        gk, (1, c, K, GADD, RANK), 0, c).astype(jnp.float32)[0]
    local = fm[..., None, :] - 1.0 + jax.random.randint(
        nk, (1, c, K, NADD, RANK), 0, 2).astype(jnp.float32)[0]
    pts = jnp.concatenate([neigh, glob, local], axis=2)
    pts = jnp.clip(pts, 0.0, float(c - 1))
    indices = pts.reshape(c, NUM_POINTS, RANK).astype(jnp.int32)

    ifl = indices.astype(jnp.float32)
    m = means[:, None, :, :]
    s = sigmas[:, None, :, :]
    inv = jnp.sqrt(1.0 / (EPS + s * s))
    diff = (ifl[:, :, None, :] - m) * inv
    dens = jnp.exp(-0.5 * jnp.sum(diff * diff, axis=-1))

    code = indices[..., 0] * c + indices[..., 1]
    eq = code[:, :, None] == code[:, None, :]
    lower = jnp.tril(jnp.ones((NUM_POINTS, NUM_POINTS), dtype=bool), -1)
    dup = jnp.any(eq & lower[None], axis=-1)
    dens = jnp.where(dup[..., None], 0.0, dens)
    dens = dens / jnp.sum(dens, axis=1, keepdims=True)
    weights = jnp.sum(dens, axis=2)
    return indices, weights


def _qk_body(q_ref, kt_ref, a_ref):
    a_ref[...] = jnp.dot(q_ref[0], kt_ref[0],
                         preferred_element_type=jnp.float32)[None]


def _qk_matmul(queries, keys):
    kt = keys.transpose(0, 2, 1)
    return pl.pallas_call(
        _qk_body,
        grid=(NH,),
        in_specs=[
            pl.BlockSpec((1, CTX, EMB), lambda h: (h, 0, 0)),
            pl.BlockSpec((1, EMB, CTX), lambda h: (h, 0, 0)),
        ],
        out_specs=pl.BlockSpec((1, CTX, CTX), lambda h: (h, 0, 0)),
        out_shape=jax.ShapeDtypeStruct((NH, CTX, CTX), jnp.float32),
    )(queries, kt)


def _pv_body(p_ref, v_ref, o_ref):
    p = p_ref[0]
    s = jnp.sum(p, axis=1, keepdims=True)
    p = p / (s + EPS)
    o_ref[...] = jnp.dot(p, v_ref[0], preferred_element_type=jnp.float32)[None]


def _pv_matmul(p, values):
    return pl.pallas_call(
        _pv_body,
        grid=(NH,),
        in_specs=[
            pl.BlockSpec((1, CTX, CTX), lambda h: (h, 0, 0)),
            pl.BlockSpec((1, CTX, EMB), lambda h: (h, 0, 0)),
        ],
        out_specs=pl.BlockSpec((1, CTX, EMB), lambda h: (h, 0, 0)),
        out_shape=jax.ShapeDtypeStruct((NH, CTX, EMB), jnp.float32),
    )(p, values)


def _sparse_attn(qkv, indices, weights):
    c = CTX
    scale = EMB ** 0.25
    q, k, v = jnp.split(qkv, 3, axis=1)

    def split_heads(t):
        return t.reshape(c, NH, EMB).transpose(1, 0, 2)

    queries = split_heads(q) / scale
    keys = split_heads(k) / scale
    values = split_heads(v)

    idx = indices.reshape(c * NUM_POINTS, RANK)
    w = weights.reshape(c * NUM_POINTS)
    rows = idx[:, 0]
    cols = idx[:, 1]
    code = rows * c + cols

    a_full = _qk_matmul(queries, keys)
    a = jnp.take(a_full.reshape(NH, c * c), code, axis=1)
    val = w[None, :] * a

    mx = jax.ops.segment_max(val.T, rows, num_segments=c)
    mx = jnp.where(jnp.isfinite(mx), mx, 0.0)
    ex = jnp.exp(val - mx[rows].T)

    p = jnp.zeros((NH, c * c), jnp.float32)
    p = p.at[jnp.arange(NH)[:, None], code[None, :]].add(ex)
    out = _pv_matmul(p.reshape(NH, c, c), values)
    return out.transpose(1, 0, 2).reshape(c, NH * EMB)


@jax.jit
def kernel(x, Wq, Wk, Wv, Wu, bu, Wp1, bp1, Wp2, bp2):
    x2d = x[0]
    qkv, params = _dense_projections(x2d, Wq, Wk, Wv, Wp1, bp1, Wp2, bp2)
    indices, weights = _gen_indices(params)
    y = _sparse_attn(qkv, indices, weights)
    out = _out_projection(y, Wu, bu)
    return out[None]
